# baseline (speedup 1.0000x reference)
import functools

import jax
import jax.numpy as jnp
from jax import lax
from jax.experimental import pallas as pl
from jax.experimental.pallas import tpu as pltpu
from jax.experimental.pallas import tpu_sc as plsc

_BUF = 100000
_FEAT = 128
_BATCH = 16384
_NC = 2
_NS = 16
_NW = _NC * _NS
_BPW = _BATCH // _NW
_CH = 128
_NCH = _BPW // _CH
_L = 16

_mesh = plsc.VectorSubcoreMesh(core_axis_name="c", subcore_axis_name="s")


@functools.partial(
    pl.kernel,
    out_type=jax.ShapeDtypeStruct((_BATCH,), jnp.int32),
    mesh=_mesh,
    scratch_types=[
        pltpu.VMEM((_BUF,), jnp.int32),
        pltpu.VMEM((_BATCH,), jnp.int32),
        pltpu.VMEM((_BPW,), jnp.int32),
        pltpu.VMEM((_BPW,), jnp.int32),
    ],
    compiler_params=pltpu.CompilerParams(needs_layout_passes=False),
)
def _winner_kernel(idx_hbm, ridx_hbm, w_hbm, tbl, idxv, ridxv, wv):
    wid = lax.axis_index("s") * _NC + lax.axis_index("c")
    base = wid * _BPW

    pltpu.sync_copy(idx_hbm, idxv)
    pltpu.sync_copy(ridx_hbm.at[pl.ds(base, _BPW)], ridxv)

    zero16 = jnp.zeros((_L,), jnp.int32)
    lane = lax.iota(jnp.int32, _L)

    def init_body(i, _):
        for u in range(10):
            tbl[pl.ds((i * 10 + u) * _L, _L)] = zero16
        return 0

    lax.fori_loop(0, _BUF // (_L * 10), init_body, 0, unroll=False)

    def scat_body(c, _):
        ind = idxv[pl.ds(c * _L, _L)]
        ival = c * _L + lane + 1
        plsc.store_scatter(tbl, [ind], ival)
        rb = plsc.load_gather(tbl, [ind])

        def fix_cond(rbc):
            return jnp.any(rbc < ival)

        def fix_once(rbc):
            plsc.store_scatter(tbl, [ind], ival, mask=rbc < ival)
            return plsc.load_gather(tbl, [ind])

        lax.while_loop(fix_cond, fix_once, rb)
        return 0

    lax.fori_loop(0, _BATCH // _L, scat_body, 0, unroll=False)

    def gath_body(c, _):
        rind = ridxv[pl.ds(c * _L, _L)]
        wv[pl.ds(c * _L, _L)] = plsc.load_gather(tbl, [rind])
        return 0

    lax.fori_loop(0, _BPW // _L, gath_body, 0, unroll=False)
    pltpu.sync_copy(wv, w_hbm.at[pl.ds(base, _BPW)])


@functools.partial(
    pl.kernel,
    out_type=jax.ShapeDtypeStruct((_BATCH, _FEAT), jnp.float32),
    mesh=_mesh,
    scratch_types=[
        pltpu.VMEM((_BPW,), jnp.int32),
        pltpu.VMEM((_BPW,), jnp.int32),
        [pltpu.VMEM((_CH,), jnp.int32) for _ in range(_NCH)],
        [pltpu.VMEM((_CH,), jnp.int32) for _ in range(_NCH)],
        [pltpu.VMEM((_CH, _FEAT), jnp.float32) for _ in range(2)],
        [pltpu.VMEM((_CH, _FEAT), jnp.float32) for _ in range(2)],
        [pltpu.SemaphoreType.DMA for _ in range(2)],
        [pltpu.SemaphoreType.DMA for _ in range(2)],
    ],
    compiler_params=pltpu.CompilerParams(needs_layout_passes=False),
)
def _row_kernel(mem_hbm, val_hbm, ridx_hbm, w_hbm, out_hbm,
                ridxv, wv, rbufs, vbufs, mrows, vrows, sem_m, sem_v):
    wid = lax.axis_index("s") * _NC + lax.axis_index("c")
    base = wid * _BPW

    pltpu.sync_copy(ridx_hbm.at[pl.ds(base, _BPW)], ridxv)
    pltpu.sync_copy(w_hbm.at[pl.ds(base, _BPW)], wv)

    for k in range(_NCH):
        def split_body(c, _, k=k):
            rind = ridxv[pl.ds(k * _CH + c * _L, _L)]
            w = wv[pl.ds(k * _CH + c * _L, _L)]
            rbufs[k][pl.ds(c * _L, _L)] = rind
            vbufs[k][pl.ds(c * _L, _L)] = jnp.maximum(w - 1, 0)
            return 0

        lax.fori_loop(0, _CH // _L, split_body, 0, unroll=False)

    def issue(k):
        cp_m = pltpu.async_copy(
            mem_hbm.at[rbufs[k]], mrows[k % 2], sem_m[k % 2])
        cp_v = pltpu.async_copy(
            val_hbm.at[vbufs[k]], vrows[k % 2], sem_v[k % 2])
        return cp_m, cp_v

    cps = issue(0)
    for k in range(_NCH):
        cps[0].wait()
        cps[1].wait()
        if k + 1 < _NCH:
            cps = issue(k + 1)
        mr = mrows[k % 2]
        vr = vrows[k % 2]

        def blend_body(c, _, k=k, mr=mr, vr=vr):
            wch = wv[pl.ds(k * _CH + c * _L, _L)]
            for rl in range(_L):
                r = c * _L + rl
                wsp = wch.at[jnp.full((_L,), rl, jnp.int32)].get(
                    mode="promise_in_bounds")
                cond = wsp > 0
                for q in range(_FEAT // _L):
                    m = mr[r, pl.ds(q * _L, _L)]
                    v = vr[r, pl.ds(q * _L, _L)]
                    mr[r, pl.ds(q * _L, _L)] = jnp.where(cond, v, m)
            return 0

        lax.fori_loop(0, _CH // _L, blend_body, 0, unroll=False)
        pltpu.sync_copy(mr, out_hbm.at[pl.ds(base + k * _CH, _CH)])


def kernel(mem, idx, val, read_idx):
    idx32 = idx.astype(jnp.int32)
    ridx32 = read_idx.astype(jnp.int32)
    w = _winner_kernel(idx32, ridx32)
    return _row_kernel(mem, val, ridx32, w)


# speedup vs baseline: 6.0083x; 6.0083x over previous
import functools

import jax
import jax.numpy as jnp
from jax import lax
from jax.experimental import pallas as pl
from jax.experimental.pallas import tpu as pltpu
from jax.experimental.pallas import tpu_sc as plsc

_BUF = 100000
_FEAT = 128
_BATCH = 16384
_NC = 2
_NS = 16
_NW = _NC * _NS
_BPW = _BATCH // _NW
_CH = 128
_NCH = _BPW // _CH
_L = 16

_mesh = plsc.VectorSubcoreMesh(core_axis_name="c", subcore_axis_name="s")


@functools.partial(
    pl.kernel,
    out_type=jax.ShapeDtypeStruct((_BATCH,), jnp.int32),
    mesh=_mesh,
    scratch_types=[
        pltpu.VMEM((_BUF,), jnp.int32),
        pltpu.VMEM((_BATCH,), jnp.int32),
        pltpu.VMEM((_BPW,), jnp.int32),
        pltpu.VMEM((_BPW,), jnp.int32),
    ],
    compiler_params=pltpu.CompilerParams(needs_layout_passes=False),
)
def _winner_kernel(idx_hbm, ridx_hbm, w_hbm, tbl, idxv, ridxv, wv):
    wid = lax.axis_index("s") * _NC + lax.axis_index("c")
    base = wid * _BPW

    pltpu.sync_copy(idx_hbm, idxv)
    pltpu.sync_copy(ridx_hbm.at[pl.ds(base, _BPW)], ridxv)

    zero16 = jnp.zeros((_L,), jnp.int32)
    lane = lax.iota(jnp.int32, _L)

    def init_body(i, _):
        for u in range(10):
            tbl[pl.ds((i * 10 + u) * _L, _L)] = zero16
        return 0

    lax.fori_loop(0, _BUF // (_L * 10), init_body, 0, unroll=False)

    def scat_body(c, _):
        ind = idxv[pl.ds(c * _L, _L)]
        ival = c * _L + lane + 1
        plsc.store_scatter(tbl, [ind], ival)
        rb = plsc.load_gather(tbl, [ind])

        def fix_cond(rbc):
            return jnp.any(rbc < ival)

        def fix_once(rbc):
            plsc.store_scatter(tbl, [ind], ival, mask=rbc < ival)
            return plsc.load_gather(tbl, [ind])

        lax.while_loop(fix_cond, fix_once, rb)
        return 0

    lax.fori_loop(0, _BATCH // _L, scat_body, 0, unroll=False)

    def gath_body(c, _):
        rind = ridxv[pl.ds(c * _L, _L)]
        wv[pl.ds(c * _L, _L)] = plsc.load_gather(tbl, [rind])
        return 0

    lax.fori_loop(0, _BPW // _L, gath_body, 0, unroll=False)
    pltpu.sync_copy(wv, w_hbm.at[pl.ds(base, _BPW)])


@functools.partial(
    pl.kernel,
    out_type=jax.ShapeDtypeStruct((_BATCH, _FEAT), jnp.float32),
    mesh=_mesh,
    scratch_types=[
        pltpu.VMEM((_BPW,), jnp.int32),
        pltpu.VMEM((_BPW,), jnp.int32),
        [pltpu.VMEM((_CH,), jnp.int32) for _ in range(_NCH)],
        [pltpu.VMEM((_CH,), jnp.int32) for _ in range(_NCH)],
        [pltpu.VMEM((_CH, _FEAT), jnp.float32) for _ in range(2)],
        [pltpu.VMEM((_CH, _FEAT), jnp.float32) for _ in range(2)],
        [pltpu.SemaphoreType.DMA for _ in range(2)],
        [pltpu.SemaphoreType.DMA for _ in range(2)],
    ],
    compiler_params=pltpu.CompilerParams(needs_layout_passes=False),
)
def _row_kernel(mem_hbm, val_hbm, ridx_hbm, w_hbm, out_hbm,
                ridxv, wv, rbufs, vbufs, mrows, vrows, sem_m, sem_v):
    wid = lax.axis_index("s") * _NC + lax.axis_index("c")
    base = wid * _BPW

    pltpu.sync_copy(ridx_hbm.at[pl.ds(base, _BPW)], ridxv)
    pltpu.sync_copy(w_hbm.at[pl.ds(base, _BPW)], wv)

    lane = lax.iota(jnp.int32, _L)
    for k in range(_NCH):
        def split_body(c, _, k=k):
            rind = ridxv[pl.ds(k * _CH + c * _L, _L)]
            w = wv[pl.ds(k * _CH + c * _L, _L)]
            pos = base + k * _CH + c * _L + lane
            rbufs[k][pl.ds(c * _L, _L)] = rind
            vbufs[k][pl.ds(c * _L, _L)] = jnp.where(w > 0, w - 1, pos)
            return 0

        lax.fori_loop(0, _CH // _L, split_body, 0, unroll=False)

    def issue(k):
        cp_m = pltpu.async_copy(
            mem_hbm.at[rbufs[k]], mrows[k % 2], sem_m[k % 2])
        cp_v = pltpu.async_copy(
            val_hbm.at[vbufs[k]], vrows[k % 2], sem_v[k % 2])
        return cp_m, cp_v

    cps = issue(0)
    for k in range(_NCH):
        cps[0].wait()
        cps[1].wait()
        if k + 1 < _NCH:
            cps = issue(k + 1)
        mr = mrows[k % 2]
        vr = vrows[k % 2]

        def blend_body(c, _, k=k, mr=mr, vr=vr):
            wch = wv[pl.ds(k * _CH + c * _L, _L)]
            for rl in range(_L):
                r = c * _L + rl
                wsp = wch.at[jnp.full((_L,), rl, jnp.int32)].get(
                    mode="promise_in_bounds")
                cond = wsp > 0
                for q in range(_FEAT // _L):
                    m = mr[r, pl.ds(q * _L, _L)]
                    v = vr[r, pl.ds(q * _L, _L)]
                    mr[r, pl.ds(q * _L, _L)] = jnp.where(cond, v, m)
            return 0

        lax.fori_loop(0, _CH // _L, blend_body, 0, unroll=False)
        pltpu.sync_copy(mr, out_hbm.at[pl.ds(base + k * _CH, _CH)])


def kernel(mem, idx, val, read_idx):
    idx32 = idx.astype(jnp.int32)
    ridx32 = read_idx.astype(jnp.int32)
    w = _winner_kernel(idx32, ridx32)
    return _row_kernel(mem, val, ridx32, w)


# speedup vs baseline: 6.0463x; 1.0063x over previous
import functools

import jax
import jax.numpy as jnp
from jax import lax
from jax.experimental import pallas as pl
from jax.experimental.pallas import tpu as pltpu
from jax.experimental.pallas import tpu_sc as plsc

_BUF = 100000
_FEAT = 128
_BATCH = 16384
_NC = 2
_NS = 16
_NW = _NC * _NS
_BPW = _BATCH // _NW
_CH = 128
_NCH = _BPW // _CH
_L = 16

_mesh = plsc.VectorSubcoreMesh(core_axis_name="c", subcore_axis_name="s")


@functools.partial(
    pl.kernel,
    out_type=jax.ShapeDtypeStruct((_BATCH,), jnp.int32),
    mesh=_mesh,
    scratch_types=[
        pltpu.VMEM((_BUF,), jnp.int32),
        pltpu.VMEM((_BATCH,), jnp.int32),
        pltpu.VMEM((_BPW,), jnp.int32),
        pltpu.VMEM((_BPW,), jnp.int32),
    ],
    compiler_params=pltpu.CompilerParams(needs_layout_passes=False),
)
def _winner_kernel(idx_hbm, ridx_hbm, w_hbm, tbl, idxv, ridxv, wv):
    wid = lax.axis_index("s") * _NC + lax.axis_index("c")
    base = wid * _BPW

    pltpu.sync_copy(idx_hbm, idxv)
    pltpu.sync_copy(ridx_hbm.at[pl.ds(base, _BPW)], ridxv)

    zero16 = jnp.zeros((_L,), jnp.int32)
    lane = lax.iota(jnp.int32, _L)

    def init_body(i, _):
        for u in range(10):
            tbl[pl.ds((i * 10 + u) * _L, _L)] = zero16
        return 0

    lax.fori_loop(0, _BUF // (_L * 10), init_body, 0, unroll=False)

    def scat_body(c, _):
        for u in range(8):
            ind = idxv[pl.ds((c * 8 + u) * _L, _L)]
            ival = (c * 8 + u) * _L + lane + 1
            plsc.store_scatter(tbl, [ind], ival)
        return 0

    lax.fori_loop(0, _BATCH // (_L * 8), scat_body, 0, unroll=False)

    def fix_body(c, _):
        for u in range(4):
            ind = idxv[pl.ds((c * 4 + u) * _L, _L)]
            ival = (c * 4 + u) * _L + lane + 1
            rb = plsc.load_gather(tbl, [ind])

            def fix_cond(rbc, ival=ival):
                return jnp.any(rbc < ival)

            def fix_once(rbc, ind=ind, ival=ival):
                plsc.store_scatter(tbl, [ind], ival, mask=rbc < ival)
                return plsc.load_gather(tbl, [ind])

            lax.while_loop(fix_cond, fix_once, rb)
        return 0

    lax.fori_loop(0, _BATCH // (_L * 4), fix_body, 0, unroll=False)

    def gath_body(c, _):
        rind = ridxv[pl.ds(c * _L, _L)]
        wv[pl.ds(c * _L, _L)] = plsc.load_gather(tbl, [rind])
        return 0

    lax.fori_loop(0, _BPW // _L, gath_body, 0, unroll=False)
    pltpu.sync_copy(wv, w_hbm.at[pl.ds(base, _BPW)])


@functools.partial(
    pl.kernel,
    out_type=jax.ShapeDtypeStruct((_BATCH, _FEAT), jnp.float32),
    mesh=_mesh,
    scratch_types=[
        pltpu.VMEM((_BPW,), jnp.int32),
        pltpu.VMEM((_BPW,), jnp.int32),
        [pltpu.VMEM((_CH,), jnp.int32) for _ in range(_NCH)],
        [pltpu.VMEM((_CH,), jnp.int32) for _ in range(_NCH)],
        [pltpu.VMEM((_CH, _FEAT), jnp.float32) for _ in range(2)],
        [pltpu.VMEM((_CH, _FEAT), jnp.float32) for _ in range(2)],
        [pltpu.SemaphoreType.DMA for _ in range(2)],
        [pltpu.SemaphoreType.DMA for _ in range(2)],
    ],
    compiler_params=pltpu.CompilerParams(needs_layout_passes=False),
)
def _row_kernel(mem_hbm, val_hbm, ridx_hbm, w_hbm, out_hbm,
                ridxv, wv, rbufs, vbufs, mrows, vrows, sem_m, sem_v):
    wid = lax.axis_index("s") * _NC + lax.axis_index("c")
    base = wid * _BPW

    pltpu.sync_copy(ridx_hbm.at[pl.ds(base, _BPW)], ridxv)
    pltpu.sync_copy(w_hbm.at[pl.ds(base, _BPW)], wv)

    lane = lax.iota(jnp.int32, _L)
    for k in range(_NCH):
        def split_body(c, _, k=k):
            rind = ridxv[pl.ds(k * _CH + c * _L, _L)]
            w = wv[pl.ds(k * _CH + c * _L, _L)]
            pos = base + k * _CH + c * _L + lane
            rbufs[k][pl.ds(c * _L, _L)] = rind
            vbufs[k][pl.ds(c * _L, _L)] = jnp.where(w > 0, w - 1, pos)
            return 0

        lax.fori_loop(0, _CH // _L, split_body, 0, unroll=False)

    def issue(k):
        cp_m = pltpu.async_copy(
            mem_hbm.at[rbufs[k]], mrows[k % 2], sem_m[k % 2])
        cp_v = pltpu.async_copy(
            val_hbm.at[vbufs[k]], vrows[k % 2], sem_v[k % 2])
        return cp_m, cp_v

    cps = issue(0)
    for k in range(_NCH):
        cps[0].wait()
        cps[1].wait()
        if k + 1 < _NCH:
            cps = issue(k + 1)
        mr = mrows[k % 2]
        vr = vrows[k % 2]

        def blend_body(c, _, k=k, mr=mr, vr=vr):
            wch = wv[pl.ds(k * _CH + c * _L, _L)]
            for rl in range(_L):
                r = c * _L + rl
                wsp = wch.at[jnp.full((_L,), rl, jnp.int32)].get(
                    mode="promise_in_bounds")
                cond = wsp > 0
                for q in range(_FEAT // _L):
                    m = mr[r, pl.ds(q * _L, _L)]
                    v = vr[r, pl.ds(q * _L, _L)]
                    mr[r, pl.ds(q * _L, _L)] = jnp.where(cond, v, m)
            return 0

        lax.fori_loop(0, _CH // _L, blend_body, 0, unroll=False)
        pltpu.sync_copy(mr, out_hbm.at[pl.ds(base + k * _CH, _CH)])


def kernel(mem, idx, val, read_idx):
    idx32 = idx.astype(jnp.int32)
    ridx32 = read_idx.astype(jnp.int32)
    w = _winner_kernel(idx32, ridx32)
    return _row_kernel(mem, val, ridx32, w)
